# baseline (speedup 1.0000x reference)
import functools

import numpy as np
import jax
import jax.numpy as jnp
from jax.experimental import pallas as pl
from jax.experimental.pallas import tpu as pltpu

WS = 8
K = 8
N = 256
C = 64
SP = WS * WS
NC = 8
E = NC * K
R = E * SP
HIGHEST = jax.lax.Precision.HIGHEST

SHIFTS = [(di, dj) for di in (-1, 0, 1) for dj in (-1, 0, 1)]


def _knn_kernel(f_ref, idx_ref, score_ref):
    f = f_ref[...]
    g = jax.lax.dot_general(f, f, (((1,), (1,)), ((), ())),
                            preferred_element_type=jnp.float32)
    sq_col = jnp.sum(f * f, axis=1, keepdims=True)
    sq_row = jnp.transpose(sq_col)
    d2 = jnp.maximum(sq_col + sq_row - 2.0 * g, 0.0)
    work = jnp.sqrt(d2 + 1e-12)
    lane = jax.lax.broadcasted_iota(jnp.int32, (N, N), 1)
    dist_cols = []
    idx_cols = []
    for _ in range(K):
        mn = jnp.min(work, axis=1, keepdims=True)
        arg = jnp.min(jnp.where(work == mn, lane, N), axis=1,
                      keepdims=True)
        dist_cols.append(mn)
        idx_cols.append(arg)
        work = jnp.where(lane == arg, jnp.inf, work)
    dist = jnp.concatenate(dist_cols, axis=1)
    idx = jnp.concatenate(idx_cols, axis=1)
    sigma = jnp.sum(dist, axis=1, keepdims=True) / K
    idx_ref[...] = idx
    score_ref[...] = jnp.exp(-dist / (sigma * sigma))


def _layer_kernel(idx_ref, score_ref, h_ref, mask_ref, wt_ref, b_ref,
                  out_ref, m_scr):
    g = pl.program_id(0)

    def gather_one(e, _):
        k = e // NC
        nn = e - k * NC
        node = g * NC + nn
        src = idx_ref[node, k]
        s = score_ref[node, k]
        m_scr[pl.ds(e * SP, SP), :] = h_ref[pl.ds(src * SP, SP), :] * s
        return 0

    jax.lax.fori_loop(0, E, gather_one, 0)

    mv = m_scr[...]
    cols = []
    for si, (di, dj) in enumerate(SHIFTS):
        s = di * WS + dj
        if s > 0:
            sh = jnp.concatenate(
                [mv[s:, :], jnp.zeros((s, C), jnp.float32)], axis=0)
        elif s < 0:
            sh = jnp.concatenate(
                [jnp.zeros((-s, C), jnp.float32), mv[:s, :]], axis=0)
        else:
            sh = mv
        cols.append(sh * mask_ref[:, si:si + 1])
    p = jnp.concatenate(cols, axis=1)
    y = jax.lax.dot_general(p, wt_ref[...], (((1,), (0,)), ((), ())),
                            precision=HIGHEST,
                            preferred_element_type=jnp.float32)
    y = y + b_ref[0:1, :]
    y = y * 0.5 * (1.0 + jax.lax.erf(y * np.float32(1.0 / np.sqrt(2.0))))
    blk = NC * SP
    acc = y[0:blk, :]
    for k in range(1, K):
        acc = jnp.maximum(acc, y[k * blk:(k + 1) * blk, :])
    out_ref[...] = acc


def _build_mask():
    r = np.arange(SP)
    i, j = r // WS, r % WS
    m = np.zeros((SP, 9), np.float32)
    for si, (di, dj) in enumerate(SHIFTS):
        m[:, si] = ((i + di >= 0) & (i + di < WS) &
                    (j + dj >= 0) & (j + dj < WS)).astype(np.float32)
    return np.tile(m, (E, 1))


@functools.partial(jax.jit, static_argnames=())
def kernel(x, w0, b0, w1, b1):
    x0 = x[0].reshape(C, 16, WS, 16, WS)
    h0 = x0.transpose(1, 3, 2, 4, 0).reshape(N * SP, C)
    feat = h0.reshape(N, SP * C)

    idx, score = pl.pallas_call(
        _knn_kernel,
        out_shape=[jax.ShapeDtypeStruct((N, K), jnp.int32),
                   jax.ShapeDtypeStruct((N, K), jnp.float32)],
    )(feat)

    mask = jnp.asarray(_build_mask())
    grid_spec = pltpu.PrefetchScalarGridSpec(
        num_scalar_prefetch=2,
        grid=(N // NC,),
        in_specs=[
            pl.BlockSpec((N * SP, C), lambda g, *_: (0, 0)),
            pl.BlockSpec((R, 9), lambda g, *_: (0, 0)),
            pl.BlockSpec((9 * C, C), lambda g, *_: (0, 0)),
            pl.BlockSpec((8, C), lambda g, *_: (0, 0)),
        ],
        out_specs=pl.BlockSpec((NC * SP, C), lambda g, *_: (g, 0)),
        scratch_shapes=[pltpu.VMEM((R, C), jnp.float32)],
    )
    layer = pl.pallas_call(
        _layer_kernel,
        grid_spec=grid_spec,
        out_shape=jax.ShapeDtypeStruct((N * SP, C), jnp.float32),
    )

    h = h0
    for (w, b) in ((w0, b0), (w1, b1)):
        wt = w.transpose(2, 3, 1, 0).reshape(9 * C, C)
        b2 = jnp.tile(b.reshape(1, C), (8, 1))
        h = layer(idx, score, h, mask, wt, b2)

    out = h.reshape(16, 16, WS, WS, C).transpose(4, 0, 2, 1, 3)
    return out.reshape(1, C, 128, 128)
